# baseline (speedup 1.0000x reference)
import functools

import jax
import jax.numpy as jnp
from jax import lax
from jax.experimental import pallas as pl
from jax.experimental.pallas import tpu as pltpu
from jax.experimental.pallas import tpu_sc as plsc

N = 10000
E = 160000
D = 128
NP = 10240
NT = 16
ROWS_PER_TILE = NP // NT
K = 128
NCH = 80
GB = 10
NGRP = NCH // GB
EPAD = NT * NCH * K


def _deg_kernel_body(ed0_hbm, ed1_hbm, zero1_hbm, ones_hbm, deg_hbm,
                     dst_v, ones_v, hist_sh):
    c = lax.axis_index("c")
    s = lax.axis_index("s")

    @pl.when(c == 0)
    def _l0():
        pltpu.sync_copy(ed0_hbm.at[1, s], dst_v)

    @pl.when(c == 1)
    def _l1():
        pltpu.sync_copy(ed1_hbm.at[1, s], dst_v)

    pltpu.sync_copy(ones_hbm, ones_v)
    pltpu.sync_copy(zero1_hbm.at[pl.ds(s * ROWS_PER_TILE, ROWS_PER_TILE)],
                    hist_sh.at[pl.ds(s * ROWS_PER_TILE, ROWS_PER_TILE)])
    plsc.subcore_barrier()

    def body(j, carry):
        g = lax.div(j, GB)
        b = lax.rem(j, GB)
        pltpu.sync_copy(ones_v.at[0], hist_sh.at[dst_v.at[g, b]], add=True)
        return carry

    lax.fori_loop(0, NCH, body, 0)
    plsc.subcore_barrier()
    pltpu.sync_copy(hist_sh.at[pl.ds(s * ROWS_PER_TILE, ROWS_PER_TILE)],
                    deg_hbm.at[c, pl.ds(s * ROWS_PER_TILE, ROWS_PER_TILE)])


def _scatter_kernel_body(hcat_hbm, ed0_hbm, ed1_hbm, zrows_hbm, out_hbm,
                         idx_v, rows0, rows1, acc_sh, sem0, sem1, semi):
    c = lax.axis_index("c")
    s = lax.axis_index("s")

    def run(h_hbm, ed_hbm):
        rows = (rows0, rows1)
        sems = (sem0, sem1)
        pltpu.sync_copy(ed_hbm.at[0, s, 0], idx_v.at[0, 0])
        pltpu.async_copy(ed_hbm.at[1, s, 0], idx_v.at[0, 1], semi)
        pltpu.async_copy(h_hbm.at[idx_v.at[0, 0, 0]], rows0, sem0)
        pltpu.sync_copy(zrows_hbm.at[pl.ds(s * ROWS_PER_TILE, ROWS_PER_TILE)],
                        acc_sh.at[pl.ds(s * ROWS_PER_TILE, ROWS_PER_TILE)])
        pltpu.make_async_copy(ed_hbm.at[1, s, 0], idx_v.at[0, 1], semi).wait()
        plsc.subcore_barrier()

        def body(g, carry):
            pg = lax.rem(g, 2)
            png = 1 - pg

            @pl.when(g < NGRP - 1)
            def _prefetch():
                pltpu.async_copy(ed_hbm.at[0, s, g + 1], idx_v.at[png, 0],
                                 semi)
                pltpu.async_copy(ed_hbm.at[1, s, g + 1], idx_v.at[png, 1],
                                 semi)

            for b in range(GB):
                rb, sb = rows[b % 2], sems[b % 2]
                nb = (b + 1) % 2
                if b < GB - 1:
                    pltpu.async_copy(h_hbm.at[idx_v.at[pg, 0, b + 1]],
                                     rows[nb], sems[nb])
                else:
                    @pl.when(g < NGRP - 1)
                    def _next_group():
                        pltpu.make_async_copy(ed_hbm.at[0, s, g + 1],
                                              idx_v.at[png, 0], semi).wait()
                        pltpu.make_async_copy(ed_hbm.at[1, s, g + 1],
                                              idx_v.at[png, 1], semi).wait()
                        pltpu.async_copy(h_hbm.at[idx_v.at[png, 0, 0]],
                                         rows[nb], sems[nb])
                pltpu.make_async_copy(h_hbm.at[idx_v.at[pg, 0, b]],
                                      rb, sb).wait()
                pltpu.sync_copy(rb, acc_sh.at[idx_v.at[pg, 1, b]], add=True)
            return carry

        lax.fori_loop(0, NGRP, body, 0)

    @pl.when(c == 0)
    def _r0():
        run(hcat_hbm.at[0], ed0_hbm)

    @pl.when(c == 1)
    def _r1():
        run(hcat_hbm.at[1], ed1_hbm)

    plsc.subcore_barrier()
    pltpu.sync_copy(acc_sh.at[pl.ds(s * ROWS_PER_TILE, ROWS_PER_TILE)],
                    out_hbm.at[c, pl.ds(s * ROWS_PER_TILE, ROWS_PER_TILE)])


@functools.lru_cache(maxsize=None)
def _sc_kernels():
    mesh = plsc.VectorSubcoreMesh(core_axis_name="c", subcore_axis_name="s")
    deg_k = pl.kernel(
        _deg_kernel_body,
        out_type=jax.ShapeDtypeStruct((2, NP), jnp.float32),
        mesh=mesh,
        scratch_types=[
            pltpu.VMEM((NGRP, GB, K), jnp.int32),
            pltpu.VMEM((GB, K), jnp.float32),
            pltpu.VMEM_SHARED((NP,), jnp.float32),
        ],
    )
    scatter_k = pl.kernel(
        _scatter_kernel_body,
        out_type=jax.ShapeDtypeStruct((2, NP, D), jnp.float32),
        mesh=mesh,
        scratch_types=[
            pltpu.VMEM((2, 2, GB, K), jnp.int32),
            pltpu.VMEM((K, D), jnp.float32),
            pltpu.VMEM((K, D), jnp.float32),
            pltpu.VMEM_SHARED((NP, D), jnp.float32),
            pltpu.SemaphoreType.DMA,
            pltpu.SemaphoreType.DMA,
            pltpu.SemaphoreType.DMA,
        ],
    )
    return deg_k, scatter_k


_BM = 2560
_GRID = 4


def _prep_body(x_ref, w0_ref, w1_ref, deg_ref, hcat_ref, dinv_ref):
    dv = lax.rsqrt(deg_ref[...] + 1.0)
    dinv_ref[...] = dv
    xb = x_ref[...]
    hcat_ref[0] = jnp.dot(xb, w0_ref[...],
                          preferred_element_type=jnp.float32) * dv[0][:, None]
    hcat_ref[1] = jnp.dot(xb, w1_ref[...],
                          preferred_element_type=jnp.float32) * dv[1][:, None]


def _combine(out_ref, hcat_ref, dinv_ref, b0_ref, b1_ref, g_ref, beta_ref):
    dv = dinv_ref[...]
    sm = (out_ref[0] + hcat_ref[0]) * dv[0][:, None] + b0_ref[...]
    sm = sm + (out_ref[1] + hcat_ref[1]) * dv[1][:, None] + b1_ref[...]
    mu = jnp.mean(sm, axis=1, keepdims=True)
    var = jnp.mean((sm - mu) ** 2, axis=1, keepdims=True)
    hn = (sm - mu) * lax.rsqrt(var + 1e-5) * g_ref[...] + beta_ref[...]
    return jnp.maximum(hn, 0.0), dv


def _mid_body(out_ref, hcat_ref, dinv_ref, b0_ref, b1_ref, g_ref, beta_ref,
              w0_ref, w1_ref, hcat2_ref):
    h, dv = _combine(out_ref, hcat_ref, dinv_ref, b0_ref, b1_ref, g_ref,
                     beta_ref)
    hcat2_ref[0] = jnp.dot(h, w0_ref[...],
                           preferred_element_type=jnp.float32) * dv[0][:, None]
    hcat2_ref[1] = jnp.dot(h, w1_ref[...],
                           preferred_element_type=jnp.float32) * dv[1][:, None]


def _final_body(out_ref, hcat_ref, dinv_ref, b0_ref, b1_ref, g_ref, beta_ref,
                h_ref):
    h, _ = _combine(out_ref, hcat_ref, dinv_ref, b0_ref, b1_ref, g_ref,
                    beta_ref)
    h_ref[...] = h


_spec_nodes = pl.BlockSpec((_BM, D), lambda i: (i, 0))
_spec_w = pl.BlockSpec((D, D), lambda i: (0, 0))
_spec_vecD = pl.BlockSpec((1, D), lambda i: (0, 0))
_spec_2n = pl.BlockSpec((2, _BM), lambda i: (0, i))
_spec_2nd = pl.BlockSpec((2, _BM, D), lambda i: (0, i, 0))

_prep_call = pl.pallas_call(
    _prep_body,
    grid=(_GRID,),
    in_specs=[_spec_nodes, _spec_w, _spec_w, _spec_2n],
    out_specs=[_spec_2nd, _spec_2n],
    out_shape=[jax.ShapeDtypeStruct((2, N, D), jnp.float32),
               jax.ShapeDtypeStruct((2, NP), jnp.float32)],
)

_mid_call = pl.pallas_call(
    _mid_body,
    grid=(_GRID,),
    in_specs=[_spec_2nd, _spec_2nd, _spec_2n, _spec_vecD, _spec_vecD,
              _spec_vecD, _spec_vecD, _spec_w, _spec_w],
    out_specs=_spec_2nd,
    out_shape=jax.ShapeDtypeStruct((2, N, D), jnp.float32),
)

_final_call = pl.pallas_call(
    _final_body,
    grid=(_GRID,),
    in_specs=[_spec_2nd, _spec_2nd, _spec_2n, _spec_vecD, _spec_vecD,
              _spec_vecD, _spec_vecD],
    out_specs=_spec_nodes,
    out_shape=jax.ShapeDtypeStruct((N, D), jnp.float32),
)


def kernel(x, edge_index_r0, edge_index_r1, W0_r0, b0_r0, W0_r1, b0_r1,
           W1_r0, b1_r0, W1_r1, b1_r1, ln0_g, ln0_b, ln1_g, ln1_b):
    f32 = jnp.float32
    pads = jnp.stack([jnp.arange(EPAD - E, dtype=jnp.int32) % N,
                      jnp.full((EPAD - E,), NP - 1, jnp.int32)])
    ed0 = jnp.concatenate([edge_index_r0, pads], axis=1).reshape(
        2, NT, NGRP, GB, K)
    ed1 = jnp.concatenate([edge_index_r1, pads], axis=1).reshape(
        2, NT, NGRP, GB, K)
    zero1 = jnp.zeros((NP,), f32)
    zrows = jnp.zeros((NP, D), f32)
    ones_b = jnp.ones((GB, K), f32)

    _deg_kernel, _scatter_kernel = _sc_kernels()
    deg = _deg_kernel(ed0, ed1, zero1, ones_b)

    hcat1, dinv = _prep_call(x, W0_r0, W0_r1, deg)
    out1 = _scatter_kernel(hcat1, ed0, ed1, zrows)
    hcat2 = _mid_call(out1, hcat1, dinv,
                      b0_r0.reshape(1, D), b0_r1.reshape(1, D),
                      ln0_g.reshape(1, D), ln0_b.reshape(1, D), W1_r0, W1_r1)
    out2 = _scatter_kernel(hcat2, ed0, ed1, zrows)
    h = _final_call(out2, hcat2, dinv,
                    b1_r0.reshape(1, D), b1_r1.reshape(1, D),
                    ln1_g.reshape(1, D), ln1_b.reshape(1, D))
    return h


# speedup vs baseline: 1.0005x; 1.0005x over previous
import functools

import jax
import jax.numpy as jnp
from jax import lax
from jax.experimental import pallas as pl
from jax.experimental.pallas import tpu as pltpu
from jax.experimental.pallas import tpu_sc as plsc

N = 10000
E = 160000
D = 128
NP = 10240
NT = 16
ROWS_PER_TILE = NP // NT
K = 128
NCH = 80
GB = 8
NGRP = NCH // GB
EPAD = NT * NCH * K


def _deg_kernel_body(ed0_hbm, ed1_hbm, zero1_hbm, ones_hbm, deg_hbm,
                     dst_v, ones_v, hist_sh):
    c = lax.axis_index("c")
    s = lax.axis_index("s")

    @pl.when(c == 0)
    def _l0():
        pltpu.sync_copy(ed0_hbm.at[1, s], dst_v)

    @pl.when(c == 1)
    def _l1():
        pltpu.sync_copy(ed1_hbm.at[1, s], dst_v)

    pltpu.sync_copy(ones_hbm, ones_v)
    pltpu.sync_copy(zero1_hbm.at[pl.ds(s * ROWS_PER_TILE, ROWS_PER_TILE)],
                    hist_sh.at[pl.ds(s * ROWS_PER_TILE, ROWS_PER_TILE)])
    plsc.subcore_barrier()

    def body(j, carry):
        g = lax.div(j, GB)
        b = lax.rem(j, GB)
        pltpu.sync_copy(ones_v.at[0], hist_sh.at[dst_v.at[g, b]], add=True)
        return carry

    lax.fori_loop(0, NCH, body, 0)
    plsc.subcore_barrier()
    pltpu.sync_copy(hist_sh.at[pl.ds(s * ROWS_PER_TILE, ROWS_PER_TILE)],
                    deg_hbm.at[c, pl.ds(s * ROWS_PER_TILE, ROWS_PER_TILE)])


def _scatter_kernel_body(hcat_hbm, ed0_hbm, ed1_hbm, zrows_hbm, out_hbm,
                         idx_v, rows0, rows1, acc_sh, sem0, sem1, semi):
    c = lax.axis_index("c")
    s = lax.axis_index("s")

    def run(h_hbm, ed_hbm):
        rows = (rows0, rows1)
        sems = (sem0, sem1)
        pltpu.sync_copy(ed_hbm.at[0, s, 0], idx_v.at[0, 0])
        pltpu.async_copy(ed_hbm.at[1, s, 0], idx_v.at[0, 1], semi)
        pltpu.async_copy(h_hbm.at[idx_v.at[0, 0, 0]], rows0, sem0)
        pltpu.sync_copy(zrows_hbm.at[pl.ds(s * ROWS_PER_TILE, ROWS_PER_TILE)],
                        acc_sh.at[pl.ds(s * ROWS_PER_TILE, ROWS_PER_TILE)])
        pltpu.make_async_copy(ed_hbm.at[1, s, 0], idx_v.at[0, 1], semi).wait()
        plsc.subcore_barrier()

        def body(g, carry):
            pg = lax.rem(g, 2)
            png = 1 - pg

            @pl.when(g < NGRP - 1)
            def _prefetch():
                pltpu.async_copy(ed_hbm.at[0, s, g + 1], idx_v.at[png, 0],
                                 semi)
                pltpu.async_copy(ed_hbm.at[1, s, g + 1], idx_v.at[png, 1],
                                 semi)

            for b in range(GB):
                rb, sb = rows[b % 2], sems[b % 2]
                nb = (b + 1) % 2
                if b < GB - 1:
                    pltpu.async_copy(h_hbm.at[idx_v.at[pg, 0, b + 1]],
                                     rows[nb], sems[nb])
                else:
                    @pl.when(g < NGRP - 1)
                    def _next_group():
                        pltpu.make_async_copy(ed_hbm.at[0, s, g + 1],
                                              idx_v.at[png, 0], semi).wait()
                        pltpu.make_async_copy(ed_hbm.at[1, s, g + 1],
                                              idx_v.at[png, 1], semi).wait()
                        pltpu.async_copy(h_hbm.at[idx_v.at[png, 0, 0]],
                                         rows[nb], sems[nb])
                pltpu.make_async_copy(h_hbm.at[idx_v.at[pg, 0, b]],
                                      rb, sb).wait()
                pltpu.sync_copy(rb, acc_sh.at[idx_v.at[pg, 1, b]], add=True)
            return carry

        lax.fori_loop(0, NGRP, body, 0)

    @pl.when(c == 0)
    def _r0():
        run(hcat_hbm.at[0], ed0_hbm)

    @pl.when(c == 1)
    def _r1():
        run(hcat_hbm.at[1], ed1_hbm)

    plsc.subcore_barrier()
    pltpu.sync_copy(acc_sh.at[pl.ds(s * ROWS_PER_TILE, ROWS_PER_TILE)],
                    out_hbm.at[c, pl.ds(s * ROWS_PER_TILE, ROWS_PER_TILE)])


@functools.lru_cache(maxsize=None)
def _sc_kernels():
    mesh = plsc.VectorSubcoreMesh(core_axis_name="c", subcore_axis_name="s")
    deg_k = pl.kernel(
        _deg_kernel_body,
        out_type=jax.ShapeDtypeStruct((2, NP), jnp.float32),
        mesh=mesh,
        scratch_types=[
            pltpu.VMEM((NGRP, GB, K), jnp.int32),
            pltpu.VMEM((GB, K), jnp.float32),
            pltpu.VMEM_SHARED((NP,), jnp.float32),
        ],
    )
    scatter_k = pl.kernel(
        _scatter_kernel_body,
        out_type=jax.ShapeDtypeStruct((2, NP, D), jnp.float32),
        mesh=mesh,
        scratch_types=[
            pltpu.VMEM((2, 2, GB, K), jnp.int32),
            pltpu.VMEM((K, D), jnp.float32),
            pltpu.VMEM((K, D), jnp.float32),
            pltpu.VMEM_SHARED((NP, D), jnp.float32),
            pltpu.SemaphoreType.DMA,
            pltpu.SemaphoreType.DMA,
            pltpu.SemaphoreType.DMA,
        ],
    )
    return deg_k, scatter_k


_BM = 2048
_GRID = 5


def _prep_body(x_ref, w0_ref, w1_ref, deg_ref, hcat_ref, dinv_ref):
    dv = lax.rsqrt(deg_ref[...] + 1.0)
    dinv_ref[...] = dv
    xb = x_ref[...]
    hcat_ref[0] = jnp.dot(xb, w0_ref[...],
                          preferred_element_type=jnp.float32) * dv[0][:, None]
    hcat_ref[1] = jnp.dot(xb, w1_ref[...],
                          preferred_element_type=jnp.float32) * dv[1][:, None]


def _combine(out_ref, hcat_ref, dinv_ref, b0_ref, b1_ref, g_ref, beta_ref):
    dv = dinv_ref[...]
    sm = (out_ref[0] + hcat_ref[0]) * dv[0][:, None] + b0_ref[...]
    sm = sm + (out_ref[1] + hcat_ref[1]) * dv[1][:, None] + b1_ref[...]
    mu = jnp.mean(sm, axis=1, keepdims=True)
    var = jnp.mean((sm - mu) ** 2, axis=1, keepdims=True)
    hn = (sm - mu) * lax.rsqrt(var + 1e-5) * g_ref[...] + beta_ref[...]
    return jnp.maximum(hn, 0.0), dv


def _mid_body(out_ref, hcat_ref, dinv_ref, b0_ref, b1_ref, g_ref, beta_ref,
              w0_ref, w1_ref, hcat2_ref):
    h, dv = _combine(out_ref, hcat_ref, dinv_ref, b0_ref, b1_ref, g_ref,
                     beta_ref)
    hcat2_ref[0] = jnp.dot(h, w0_ref[...],
                           preferred_element_type=jnp.float32) * dv[0][:, None]
    hcat2_ref[1] = jnp.dot(h, w1_ref[...],
                           preferred_element_type=jnp.float32) * dv[1][:, None]


def _final_body(out_ref, hcat_ref, dinv_ref, b0_ref, b1_ref, g_ref, beta_ref,
                h_ref):
    h, _ = _combine(out_ref, hcat_ref, dinv_ref, b0_ref, b1_ref, g_ref,
                    beta_ref)
    h_ref[...] = h


_spec_nodes = pl.BlockSpec((_BM, D), lambda i: (i, 0))
_spec_w = pl.BlockSpec((D, D), lambda i: (0, 0))
_spec_vecD = pl.BlockSpec((1, D), lambda i: (0, 0))
_spec_2n = pl.BlockSpec((2, _BM), lambda i: (0, i))
_spec_2nd = pl.BlockSpec((2, _BM, D), lambda i: (0, i, 0))

_prep_call = pl.pallas_call(
    _prep_body,
    grid=(_GRID,),
    in_specs=[_spec_nodes, _spec_w, _spec_w, _spec_2n],
    out_specs=[_spec_2nd, _spec_2n],
    out_shape=[jax.ShapeDtypeStruct((2, N, D), jnp.float32),
               jax.ShapeDtypeStruct((2, NP), jnp.float32)],
)

_mid_call = pl.pallas_call(
    _mid_body,
    grid=(_GRID,),
    in_specs=[_spec_2nd, _spec_2nd, _spec_2n, _spec_vecD, _spec_vecD,
              _spec_vecD, _spec_vecD, _spec_w, _spec_w],
    out_specs=_spec_2nd,
    out_shape=jax.ShapeDtypeStruct((2, N, D), jnp.float32),
)

_final_call = pl.pallas_call(
    _final_body,
    grid=(_GRID,),
    in_specs=[_spec_2nd, _spec_2nd, _spec_2n, _spec_vecD, _spec_vecD,
              _spec_vecD, _spec_vecD],
    out_specs=_spec_nodes,
    out_shape=jax.ShapeDtypeStruct((N, D), jnp.float32),
)


def kernel(x, edge_index_r0, edge_index_r1, W0_r0, b0_r0, W0_r1, b0_r1,
           W1_r0, b1_r0, W1_r1, b1_r1, ln0_g, ln0_b, ln1_g, ln1_b):
    f32 = jnp.float32
    pads = jnp.stack([jnp.arange(EPAD - E, dtype=jnp.int32) % N,
                      jnp.full((EPAD - E,), NP - 1, jnp.int32)])
    ed0 = jnp.concatenate([edge_index_r0, pads], axis=1).reshape(
        2, NT, NGRP, GB, K)
    ed1 = jnp.concatenate([edge_index_r1, pads], axis=1).reshape(
        2, NT, NGRP, GB, K)
    zero1 = jnp.zeros((NP,), f32)
    zrows = jnp.zeros((NP, D), f32)
    ones_b = jnp.ones((GB, K), f32)

    _deg_kernel, _scatter_kernel = _sc_kernels()
    deg = _deg_kernel(ed0, ed1, zero1, ones_b)

    hcat1, dinv = _prep_call(x, W0_r0, W0_r1, deg)
    out1 = _scatter_kernel(hcat1, ed0, ed1, zrows)
    hcat2 = _mid_call(out1, hcat1, dinv,
                      b0_r0.reshape(1, D), b0_r1.reshape(1, D),
                      ln0_g.reshape(1, D), ln0_b.reshape(1, D), W1_r0, W1_r1)
    out2 = _scatter_kernel(hcat2, ed0, ed1, zrows)
    h = _final_call(out2, hcat2, dinv,
                    b1_r0.reshape(1, D), b1_r1.reshape(1, D),
                    ln1_g.reshape(1, D), ln1_b.reshape(1, D))
    return h


# speedup vs baseline: 1.0267x; 1.0262x over previous
import functools

import jax
import jax.numpy as jnp
from jax import lax
from jax.experimental import pallas as pl
from jax.experimental.pallas import tpu as pltpu
from jax.experimental.pallas import tpu_sc as plsc

N = 10000
E = 160000
D = 128
NP = 10240
NT = 16
ROWS_PER_TILE = NP // NT
K = 128
NCH = 80
GB = 8
NGRP = NCH // GB
EPAD = NT * NCH * K


def _deg_kernel_body(ed0_hbm, ed1_hbm, zero1_hbm, ones_hbm, deg_hbm,
                     dst_v, ones_v, hist_sh):
    c = lax.axis_index("c")
    s = lax.axis_index("s")

    @pl.when(c == 0)
    def _l0():
        pltpu.sync_copy(ed0_hbm.at[1, s], dst_v)

    @pl.when(c == 1)
    def _l1():
        pltpu.sync_copy(ed1_hbm.at[1, s], dst_v)

    pltpu.sync_copy(ones_hbm, ones_v)
    pltpu.sync_copy(zero1_hbm.at[pl.ds(s * ROWS_PER_TILE, ROWS_PER_TILE)],
                    hist_sh.at[pl.ds(s * ROWS_PER_TILE, ROWS_PER_TILE)])
    plsc.subcore_barrier()

    def body(j, carry):
        g = lax.div(j, GB)
        b = lax.rem(j, GB)
        pltpu.sync_copy(ones_v.at[0], hist_sh.at[dst_v.at[g, b]], add=True)
        return carry

    lax.fori_loop(0, NCH, body, 0)
    plsc.subcore_barrier()
    pltpu.sync_copy(hist_sh.at[pl.ds(s * ROWS_PER_TILE, ROWS_PER_TILE)],
                    deg_hbm.at[c, pl.ds(s * ROWS_PER_TILE, ROWS_PER_TILE)])


def _scatter_kernel_body(hcat_hbm, ed0_hbm, ed1_hbm, out_hbm,
                         idx_v, rows0, rows1, acc_sh, sem0, sem1, semi):
    c = lax.axis_index("c")
    s = lax.axis_index("s")

    def run(h_hbm, ed_hbm):
        rows = (rows0, rows1)
        sems = (sem0, sem1)
        pltpu.sync_copy(ed_hbm.at[0, s, 0], idx_v.at[0, 0])
        pltpu.async_copy(ed_hbm.at[1, s, 0], idx_v.at[0, 1], semi)
        pltpu.async_copy(h_hbm.at[idx_v.at[0, 0, 0]], rows0, sem0)

        def zbody(j, carry):
            for i in range(D // 16):
                rows1[j, pl.ds(i * 16, 16)] = jnp.zeros((16,), jnp.float32)
            return carry

        lax.fori_loop(0, K, zbody, 0)
        for t in range(ROWS_PER_TILE // K):
            pltpu.sync_copy(
                rows1, acc_sh.at[pl.ds(s * ROWS_PER_TILE + t * K, K)])
        pltpu.make_async_copy(ed_hbm.at[1, s, 0], idx_v.at[0, 1], semi).wait()
        plsc.subcore_barrier()

        def body(g, carry):
            pg = lax.rem(g, 2)
            png = 1 - pg

            @pl.when(g < NGRP - 1)
            def _prefetch():
                pltpu.async_copy(ed_hbm.at[0, s, g + 1], idx_v.at[png, 0],
                                 semi)
                pltpu.async_copy(ed_hbm.at[1, s, g + 1], idx_v.at[png, 1],
                                 semi)

            for b in range(GB):
                rb, sb = rows[b % 2], sems[b % 2]
                nb = (b + 1) % 2
                if b < GB - 1:
                    pltpu.async_copy(h_hbm.at[idx_v.at[pg, 0, b + 1]],
                                     rows[nb], sems[nb])
                else:
                    @pl.when(g < NGRP - 1)
                    def _next_group():
                        pltpu.make_async_copy(ed_hbm.at[0, s, g + 1],
                                              idx_v.at[png, 0], semi).wait()
                        pltpu.make_async_copy(ed_hbm.at[1, s, g + 1],
                                              idx_v.at[png, 1], semi).wait()
                        pltpu.async_copy(h_hbm.at[idx_v.at[png, 0, 0]],
                                         rows[nb], sems[nb])
                pltpu.make_async_copy(h_hbm.at[idx_v.at[pg, 0, b]],
                                      rb, sb).wait()
                pltpu.sync_copy(rb, acc_sh.at[idx_v.at[pg, 1, b]], add=True)
            return carry

        lax.fori_loop(0, NGRP, body, 0)

    @pl.when(c == 0)
    def _r0():
        run(hcat_hbm.at[0], ed0_hbm)

    @pl.when(c == 1)
    def _r1():
        run(hcat_hbm.at[1], ed1_hbm)

    plsc.subcore_barrier()
    pltpu.sync_copy(acc_sh.at[pl.ds(s * ROWS_PER_TILE, ROWS_PER_TILE)],
                    out_hbm.at[c, pl.ds(s * ROWS_PER_TILE, ROWS_PER_TILE)])


@functools.lru_cache(maxsize=None)
def _sc_kernels():
    mesh = plsc.VectorSubcoreMesh(core_axis_name="c", subcore_axis_name="s")
    deg_k = pl.kernel(
        _deg_kernel_body,
        out_type=jax.ShapeDtypeStruct((2, NP), jnp.float32),
        mesh=mesh,
        scratch_types=[
            pltpu.VMEM((NGRP, GB, K), jnp.int32),
            pltpu.VMEM((GB, K), jnp.float32),
            pltpu.VMEM_SHARED((NP,), jnp.float32),
        ],
    )
    scatter_k = pl.kernel(
        _scatter_kernel_body,
        out_type=jax.ShapeDtypeStruct((2, NP, D), jnp.float32),
        mesh=mesh,
        scratch_types=[
            pltpu.VMEM((2, 2, GB, K), jnp.int32),
            pltpu.VMEM((K, D), jnp.float32),
            pltpu.VMEM((K, D), jnp.float32),
            pltpu.VMEM_SHARED((NP, D), jnp.float32),
            pltpu.SemaphoreType.DMA,
            pltpu.SemaphoreType.DMA,
            pltpu.SemaphoreType.DMA,
        ],
    )
    return deg_k, scatter_k


_BM = 2048
_GRID = 5


def _prep_body(x_ref, w0_ref, w1_ref, deg_ref, hcat_ref, dinv_ref):
    dv = lax.rsqrt(deg_ref[...] + 1.0)
    dinv_ref[...] = dv
    xb = x_ref[...]
    hcat_ref[0] = jnp.dot(xb, w0_ref[...],
                          preferred_element_type=jnp.float32) * dv[0][:, None]
    hcat_ref[1] = jnp.dot(xb, w1_ref[...],
                          preferred_element_type=jnp.float32) * dv[1][:, None]


def _combine(out_ref, hcat_ref, dinv_ref, b0_ref, b1_ref, g_ref, beta_ref):
    dv = dinv_ref[...]
    sm = (out_ref[0] + hcat_ref[0]) * dv[0][:, None] + b0_ref[...]
    sm = sm + (out_ref[1] + hcat_ref[1]) * dv[1][:, None] + b1_ref[...]
    mu = jnp.mean(sm, axis=1, keepdims=True)
    var = jnp.mean((sm - mu) ** 2, axis=1, keepdims=True)
    hn = (sm - mu) * lax.rsqrt(var + 1e-5) * g_ref[...] + beta_ref[...]
    return jnp.maximum(hn, 0.0), dv


def _mid_body(out_ref, hcat_ref, dinv_ref, b0_ref, b1_ref, g_ref, beta_ref,
              w0_ref, w1_ref, hcat2_ref):
    h, dv = _combine(out_ref, hcat_ref, dinv_ref, b0_ref, b1_ref, g_ref,
                     beta_ref)
    hcat2_ref[0] = jnp.dot(h, w0_ref[...],
                           preferred_element_type=jnp.float32) * dv[0][:, None]
    hcat2_ref[1] = jnp.dot(h, w1_ref[...],
                           preferred_element_type=jnp.float32) * dv[1][:, None]


def _final_body(out_ref, hcat_ref, dinv_ref, b0_ref, b1_ref, g_ref, beta_ref,
                h_ref):
    h, _ = _combine(out_ref, hcat_ref, dinv_ref, b0_ref, b1_ref, g_ref,
                    beta_ref)
    h_ref[...] = h


_spec_nodes = pl.BlockSpec((_BM, D), lambda i: (i, 0))
_spec_w = pl.BlockSpec((D, D), lambda i: (0, 0))
_spec_vecD = pl.BlockSpec((1, D), lambda i: (0, 0))
_spec_2n = pl.BlockSpec((2, _BM), lambda i: (0, i))
_spec_2nd = pl.BlockSpec((2, _BM, D), lambda i: (0, i, 0))

_prep_call = pl.pallas_call(
    _prep_body,
    grid=(_GRID,),
    in_specs=[_spec_nodes, _spec_w, _spec_w, _spec_2n],
    out_specs=[_spec_2nd, _spec_2n],
    out_shape=[jax.ShapeDtypeStruct((2, N, D), jnp.float32),
               jax.ShapeDtypeStruct((2, NP), jnp.float32)],
)

_mid_call = pl.pallas_call(
    _mid_body,
    grid=(_GRID,),
    in_specs=[_spec_2nd, _spec_2nd, _spec_2n, _spec_vecD, _spec_vecD,
              _spec_vecD, _spec_vecD, _spec_w, _spec_w],
    out_specs=_spec_2nd,
    out_shape=jax.ShapeDtypeStruct((2, N, D), jnp.float32),
)

_final_call = pl.pallas_call(
    _final_body,
    grid=(_GRID,),
    in_specs=[_spec_2nd, _spec_2nd, _spec_2n, _spec_vecD, _spec_vecD,
              _spec_vecD, _spec_vecD],
    out_specs=_spec_nodes,
    out_shape=jax.ShapeDtypeStruct((N, D), jnp.float32),
)


def kernel(x, edge_index_r0, edge_index_r1, W0_r0, b0_r0, W0_r1, b0_r1,
           W1_r0, b1_r0, W1_r1, b1_r1, ln0_g, ln0_b, ln1_g, ln1_b):
    f32 = jnp.float32
    pads = jnp.stack([jnp.arange(EPAD - E, dtype=jnp.int32) % N,
                      jnp.full((EPAD - E,), NP - 1, jnp.int32)])
    ed0 = jnp.concatenate([edge_index_r0, pads], axis=1).reshape(
        2, NT, NGRP, GB, K)
    ed1 = jnp.concatenate([edge_index_r1, pads], axis=1).reshape(
        2, NT, NGRP, GB, K)
    zero1 = jnp.zeros((NP,), f32)
    ones_b = jnp.ones((GB, K), f32)

    _deg_kernel, _scatter_kernel = _sc_kernels()
    deg = _deg_kernel(ed0, ed1, zero1, ones_b)

    hcat1, dinv = _prep_call(x, W0_r0, W0_r1, deg)
    out1 = _scatter_kernel(hcat1, ed0, ed1)
    hcat2 = _mid_call(out1, hcat1, dinv,
                      b0_r0.reshape(1, D), b0_r1.reshape(1, D),
                      ln0_g.reshape(1, D), ln0_b.reshape(1, D), W1_r0, W1_r1)
    out2 = _scatter_kernel(hcat2, ed0, ed1)
    h = _final_call(out2, hcat2, dinv,
                    b1_r0.reshape(1, D), b1_r1.reshape(1, D),
                    ln1_g.reshape(1, D), ln1_b.reshape(1, D))
    return h


# speedup vs baseline: 1.0323x; 1.0055x over previous
import functools

import jax
import jax.numpy as jnp
from jax import lax
from jax.experimental import pallas as pl
from jax.experimental.pallas import tpu as pltpu
from jax.experimental.pallas import tpu_sc as plsc

N = 10000
E = 160000
D = 128
NP = 10240
NT = 16
ROWS_PER_TILE = NP // NT
K = 128
NCH = 80
GB = 8
NGRP = NCH // GB
EPAD = NT * NCH * K


def _deg_kernel_body(ed0_hbm, ed1_hbm, zero1_hbm, ones_hbm, deg_hbm,
                     dst_v, ones_v, hist_sh):
    c = lax.axis_index("c")
    s = lax.axis_index("s")

    @pl.when(c == 0)
    def _l0():
        pltpu.sync_copy(ed0_hbm.at[1, s], dst_v)

    @pl.when(c == 1)
    def _l1():
        pltpu.sync_copy(ed1_hbm.at[1, s], dst_v)

    pltpu.sync_copy(ones_hbm, ones_v)
    pltpu.sync_copy(zero1_hbm.at[pl.ds(s * ROWS_PER_TILE, ROWS_PER_TILE)],
                    hist_sh.at[pl.ds(s * ROWS_PER_TILE, ROWS_PER_TILE)])
    plsc.subcore_barrier()

    def body(j, carry):
        g = lax.div(j, GB)
        b = lax.rem(j, GB)
        pltpu.sync_copy(ones_v.at[0], hist_sh.at[dst_v.at[g, b]], add=True)
        return carry

    lax.fori_loop(0, NCH, body, 0)
    plsc.subcore_barrier()
    pltpu.sync_copy(hist_sh.at[pl.ds(s * ROWS_PER_TILE, ROWS_PER_TILE)],
                    deg_hbm.at[c, pl.ds(s * ROWS_PER_TILE, ROWS_PER_TILE)])


def _scatter_kernel_body(hcat_hbm, ed0_hbm, ed1_hbm, out_hbm,
                         idx_v, rows0, rows1, acc_sh, sem0, sem1, semi, ssem):
    c = lax.axis_index("c")
    s = lax.axis_index("s")

    def run(h_hbm, ed_hbm):
        rows = (rows0, rows1)
        sems = (sem0, sem1)

        def _wait_scatter():
            pltpu.make_async_copy(rows0, acc_sh.at[idx_v.at[0, 1, 0]],
                                  ssem).wait()

        pltpu.sync_copy(ed_hbm.at[0, s, 0], idx_v.at[0, 0])
        pltpu.async_copy(ed_hbm.at[1, s, 0], idx_v.at[0, 1], semi)
        pltpu.async_copy(h_hbm.at[idx_v.at[0, 0, 0]], rows0, sem0)

        def zbody(j, carry):
            for i in range(D // 16):
                rows1[j, pl.ds(i * 16, 16)] = jnp.zeros((16,), jnp.float32)
            return carry

        lax.fori_loop(0, K, zbody, 0)
        for t in range(ROWS_PER_TILE // K):
            pltpu.sync_copy(
                rows1, acc_sh.at[pl.ds(s * ROWS_PER_TILE + t * K, K)])
        pltpu.make_async_copy(ed_hbm.at[1, s, 0], idx_v.at[0, 1], semi).wait()
        plsc.subcore_barrier()

        def body(g, carry):
            pg = lax.rem(g, 2)
            png = 1 - pg

            @pl.when(g < NGRP - 1)
            def _prefetch():
                pltpu.async_copy(ed_hbm.at[0, s, g + 1], idx_v.at[png, 0],
                                 semi)
                pltpu.async_copy(ed_hbm.at[1, s, g + 1], idx_v.at[png, 1],
                                 semi)

            for b in range(GB):
                rb, sb = rows[b % 2], sems[b % 2]
                nb = (b + 1) % 2
                if b == 0:
                    @pl.when(g > 0)
                    def _ws0():
                        _wait_scatter()
                else:
                    _wait_scatter()
                if b < GB - 1:
                    pltpu.async_copy(h_hbm.at[idx_v.at[pg, 0, b + 1]],
                                     rows[nb], sems[nb])
                else:
                    @pl.when(g < NGRP - 1)
                    def _next_group():
                        pltpu.make_async_copy(ed_hbm.at[0, s, g + 1],
                                              idx_v.at[png, 0], semi).wait()
                        pltpu.make_async_copy(ed_hbm.at[1, s, g + 1],
                                              idx_v.at[png, 1], semi).wait()
                        pltpu.async_copy(h_hbm.at[idx_v.at[png, 0, 0]],
                                         rows[nb], sems[nb])
                pltpu.make_async_copy(h_hbm.at[idx_v.at[pg, 0, b]],
                                      rb, sb).wait()
                pltpu.async_copy(rb, acc_sh.at[idx_v.at[pg, 1, b]], ssem,
                                 add=True)
            return carry

        lax.fori_loop(0, NGRP, body, 0)
        _wait_scatter()

    @pl.when(c == 0)
    def _r0():
        run(hcat_hbm.at[0], ed0_hbm)

    @pl.when(c == 1)
    def _r1():
        run(hcat_hbm.at[1], ed1_hbm)

    plsc.subcore_barrier()
    pltpu.sync_copy(acc_sh.at[pl.ds(s * ROWS_PER_TILE, ROWS_PER_TILE)],
                    out_hbm.at[c, pl.ds(s * ROWS_PER_TILE, ROWS_PER_TILE)])


@functools.lru_cache(maxsize=None)
def _sc_kernels():
    mesh = plsc.VectorSubcoreMesh(core_axis_name="c", subcore_axis_name="s")
    deg_k = pl.kernel(
        _deg_kernel_body,
        out_type=jax.ShapeDtypeStruct((2, NP), jnp.float32),
        mesh=mesh,
        scratch_types=[
            pltpu.VMEM((NGRP, GB, K), jnp.int32),
            pltpu.VMEM((GB, K), jnp.float32),
            pltpu.VMEM_SHARED((NP,), jnp.float32),
        ],
    )
    scatter_k = pl.kernel(
        _scatter_kernel_body,
        out_type=jax.ShapeDtypeStruct((2, NP, D), jnp.float32),
        mesh=mesh,
        scratch_types=[
            pltpu.VMEM((2, 2, GB, K), jnp.int32),
            pltpu.VMEM((K, D), jnp.float32),
            pltpu.VMEM((K, D), jnp.float32),
            pltpu.VMEM_SHARED((NP, D), jnp.float32),
            pltpu.SemaphoreType.DMA,
            pltpu.SemaphoreType.DMA,
            pltpu.SemaphoreType.DMA,
            pltpu.SemaphoreType.DMA,
        ],
    )
    return deg_k, scatter_k


_BM = 2048
_GRID = 5


def _prep_body(x_ref, w0_ref, w1_ref, deg_ref, hcat_ref, dinv_ref):
    dv = lax.rsqrt(deg_ref[...] + 1.0)
    dinv_ref[...] = dv
    xb = x_ref[...]
    hcat_ref[0] = jnp.dot(xb, w0_ref[...],
                          preferred_element_type=jnp.float32) * dv[0][:, None]
    hcat_ref[1] = jnp.dot(xb, w1_ref[...],
                          preferred_element_type=jnp.float32) * dv[1][:, None]


def _combine(out_ref, hcat_ref, dinv_ref, b0_ref, b1_ref, g_ref, beta_ref):
    dv = dinv_ref[...]
    sm = (out_ref[0] + hcat_ref[0]) * dv[0][:, None] + b0_ref[...]
    sm = sm + (out_ref[1] + hcat_ref[1]) * dv[1][:, None] + b1_ref[...]
    mu = jnp.mean(sm, axis=1, keepdims=True)
    var = jnp.mean((sm - mu) ** 2, axis=1, keepdims=True)
    hn = (sm - mu) * lax.rsqrt(var + 1e-5) * g_ref[...] + beta_ref[...]
    return jnp.maximum(hn, 0.0), dv


def _mid_body(out_ref, hcat_ref, dinv_ref, b0_ref, b1_ref, g_ref, beta_ref,
              w0_ref, w1_ref, hcat2_ref):
    h, dv = _combine(out_ref, hcat_ref, dinv_ref, b0_ref, b1_ref, g_ref,
                     beta_ref)
    hcat2_ref[0] = jnp.dot(h, w0_ref[...],
                           preferred_element_type=jnp.float32) * dv[0][:, None]
    hcat2_ref[1] = jnp.dot(h, w1_ref[...],
                           preferred_element_type=jnp.float32) * dv[1][:, None]


def _final_body(out_ref, hcat_ref, dinv_ref, b0_ref, b1_ref, g_ref, beta_ref,
                h_ref):
    h, _ = _combine(out_ref, hcat_ref, dinv_ref, b0_ref, b1_ref, g_ref,
                    beta_ref)
    h_ref[...] = h


_spec_nodes = pl.BlockSpec((_BM, D), lambda i: (i, 0))
_spec_w = pl.BlockSpec((D, D), lambda i: (0, 0))
_spec_vecD = pl.BlockSpec((1, D), lambda i: (0, 0))
_spec_2n = pl.BlockSpec((2, _BM), lambda i: (0, i))
_spec_2nd = pl.BlockSpec((2, _BM, D), lambda i: (0, i, 0))

_prep_call = pl.pallas_call(
    _prep_body,
    grid=(_GRID,),
    in_specs=[_spec_nodes, _spec_w, _spec_w, _spec_2n],
    out_specs=[_spec_2nd, _spec_2n],
    out_shape=[jax.ShapeDtypeStruct((2, N, D), jnp.float32),
               jax.ShapeDtypeStruct((2, NP), jnp.float32)],
)

_mid_call = pl.pallas_call(
    _mid_body,
    grid=(_GRID,),
    in_specs=[_spec_2nd, _spec_2nd, _spec_2n, _spec_vecD, _spec_vecD,
              _spec_vecD, _spec_vecD, _spec_w, _spec_w],
    out_specs=_spec_2nd,
    out_shape=jax.ShapeDtypeStruct((2, N, D), jnp.float32),
)

_final_call = pl.pallas_call(
    _final_body,
    grid=(_GRID,),
    in_specs=[_spec_2nd, _spec_2nd, _spec_2n, _spec_vecD, _spec_vecD,
              _spec_vecD, _spec_vecD],
    out_specs=_spec_nodes,
    out_shape=jax.ShapeDtypeStruct((N, D), jnp.float32),
)


def kernel(x, edge_index_r0, edge_index_r1, W0_r0, b0_r0, W0_r1, b0_r1,
           W1_r0, b1_r0, W1_r1, b1_r1, ln0_g, ln0_b, ln1_g, ln1_b):
    f32 = jnp.float32
    pads = jnp.stack([jnp.arange(EPAD - E, dtype=jnp.int32) % N,
                      jnp.full((EPAD - E,), NP - 1, jnp.int32)])
    ed0 = jnp.concatenate([edge_index_r0, pads], axis=1).reshape(
        2, NT, NGRP, GB, K)
    ed1 = jnp.concatenate([edge_index_r1, pads], axis=1).reshape(
        2, NT, NGRP, GB, K)
    zero1 = jnp.zeros((NP,), f32)
    ones_b = jnp.ones((GB, K), f32)

    _deg_kernel, _scatter_kernel = _sc_kernels()
    deg = _deg_kernel(ed0, ed1, zero1, ones_b)

    hcat1, dinv = _prep_call(x, W0_r0, W0_r1, deg)
    out1 = _scatter_kernel(hcat1, ed0, ed1)
    hcat2 = _mid_call(out1, hcat1, dinv,
                      b0_r0.reshape(1, D), b0_r1.reshape(1, D),
                      ln0_g.reshape(1, D), ln0_b.reshape(1, D), W1_r0, W1_r1)
    out2 = _scatter_kernel(hcat2, ed0, ed1)
    h = _final_call(out2, hcat2, dinv,
                    b1_r0.reshape(1, D), b1_r1.reshape(1, D),
                    ln1_g.reshape(1, D), ln1_b.reshape(1, D))
    return h
